# baseline (speedup 1.0000x reference)
import functools

import jax
import jax.numpy as jnp
from jax.experimental import pallas as pl
from jax.experimental.pallas import tpu as pltpu

N = 131072
C = 256
NUM_CLASSES = 20
BLK = 2048


def _mm(a, b):
    return jax.lax.dot_general(
        a, b, (((a.ndim - 1,), (0,)), ((), ())),
        preferred_element_type=jnp.float32)


def _pass1_kernel(feat_ref, seg_ref, wb1_ref, bb1_ref, wb2_ref, bb2_ref,
                  wseg_ref, bseg_ref,
                  h_ref, sumh_ref, g_ref, segnll_ref, vcnt_ref):
    i = pl.program_id(0)

    @pl.when(i == 0)
    def _init():
        sumh_ref[...] = jnp.zeros_like(sumh_ref)
        g_ref[...] = jnp.zeros_like(g_ref)
        segnll_ref[...] = jnp.zeros_like(segnll_ref)
        vcnt_ref[...] = jnp.zeros_like(vcnt_ref)

    x = feat_ref[...]
    h1 = jnp.maximum(_mm(x, wb1_ref[...]) + bb1_ref[...], 0.0)
    h = jnp.maximum(_mm(h1, wb2_ref[...]) + bb2_ref[...], 0.0)
    h_ref[...] = h

    sumh_ref[...] += jnp.sum(h, axis=0, keepdims=True)
    g_ref[...] += jax.lax.dot_general(
        h, h, (((0,), (0,)), ((), ())), preferred_element_type=jnp.float32)

    logits = _mm(h, wseg_ref[...]) + bseg_ref[...]
    m = jnp.max(logits, axis=-1, keepdims=True)
    lse = m + jnp.log(jnp.sum(jnp.exp(logits - m), axis=-1, keepdims=True))
    seg = seg_ref[...]
    seg_c = jnp.clip(seg, 0, NUM_CLASSES - 1)
    cols = jax.lax.broadcasted_iota(jnp.int32, logits.shape, 1)
    picked = jnp.sum(jnp.where(cols == seg_c, logits, 0.0), axis=-1,
                     keepdims=True)
    nll = lse - picked
    valid = (seg != -1).astype(jnp.float32)
    segnll_ref[...] += jnp.sum(nll * valid).reshape(1, 1)
    vcnt_ref[...] += jnp.sum(valid).reshape(1, 1)


def _fold(sum_h, g_mat, w, b, gam, bet, n):
    sw = _mm(sum_h, w)
    mu = sw / n + b
    t = _mm(g_mat, w)
    ssq = jnp.sum(w * t, axis=0, keepdims=True)
    ey2 = (ssq + 2.0 * b * sw) / n + b * b
    var = ey2 - mu * mu
    scale = gam * jax.lax.rsqrt(var + 1e-3)
    return w * scale, (b - mu) * scale + bet


def _pass2_kernel(h_ref, coord_ref, cent_ref, inst_ref, sumh_ref, g_ref,
                  wxy1_ref, bxy1_ref, gxy_ref, bexy_ref, wxy2_ref, bxy2_ref,
                  wz1_ref, bz1_ref, gz_ref, bez_ref, wz2_ref, bz2_ref,
                  l1xy_ref, cosxy_ref, l1z_ref, cosz_ref, cnt_ref,
                  wfxy_ref, cxy_ref, wfz_ref, cz_ref):
    i = pl.program_id(0)
    n = jnp.float32(N)

    @pl.when(i == 0)
    def _init():
        wf, c = _fold(sumh_ref[...], g_ref[...], wxy1_ref[...], bxy1_ref[...],
                      gxy_ref[...], bexy_ref[...], n)
        wfxy_ref[...] = wf
        cxy_ref[...] = c
        wf, c = _fold(sumh_ref[...], g_ref[...], wz1_ref[...], bz1_ref[...],
                      gz_ref[...], bez_ref[...], n)
        wfz_ref[...] = wf
        cz_ref[...] = c
        l1xy_ref[...] = jnp.zeros_like(l1xy_ref)
        cosxy_ref[...] = jnp.zeros_like(cosxy_ref)
        l1z_ref[...] = jnp.zeros_like(l1z_ref)
        cosz_ref[...] = jnp.zeros_like(cosz_ref)
        cnt_ref[...] = jnp.zeros_like(cnt_ref)

    h = h_ref[...]
    axy = jnp.maximum(_mm(h, wfxy_ref[...]) + cxy_ref[...], 0.0)
    pxy = _mm(axy, wxy2_ref[...]) + bxy2_ref[...]
    az = jnp.maximum(_mm(h, wfz_ref[...]) + cz_ref[...], 0.0)
    pz = _mm(az, wz2_ref[...]) + bz2_ref[...]

    gt = cent_ref[...] - coord_ref[...]
    gxy = gt[:, 0:2]
    gz = gt[:, 2:3]
    mask = (inst_ref[...] != -1).astype(jnp.float32)

    l1xy = jnp.sum(jnp.abs(pxy - gxy), axis=-1, keepdims=True)
    pn = pxy / (jnp.sqrt(jnp.sum(pxy * pxy, axis=-1, keepdims=True)) + 1e-8)
    gn = gxy / (jnp.sqrt(jnp.sum(gxy * gxy, axis=-1, keepdims=True)) + 1e-8)
    cxy = -jnp.sum(pn * gn, axis=-1, keepdims=True)

    l1z = jnp.abs(pz - gz)
    czv = -(pz / (jnp.abs(pz) + 1e-8)) * (gz / (jnp.abs(gz) + 1e-8))

    l1xy_ref[...] += jnp.sum(l1xy * mask).reshape(1, 1)
    cosxy_ref[...] += jnp.sum(cxy * mask).reshape(1, 1)
    l1z_ref[...] += jnp.sum(l1z * mask).reshape(1, 1)
    cosz_ref[...] += jnp.sum(czv * mask).reshape(1, 1)
    cnt_ref[...] += jnp.sum(mask).reshape(1, 1)


def _full(shape):
    return pl.BlockSpec(shape, lambda i: tuple(0 for _ in shape))


@jax.jit
def kernel(coord, feat, segment, instance, instance_centroid,
           W_b1, b_b1, W_b2, b_b2,
           W_xy1, b_xy1, g_xy, be_xy, W_xy2, b_xy2,
           W_z1, b_z1, g_z, be_z, W_z2, b_z2,
           W_seg, b_seg):
    f32 = jnp.float32
    nblk = N // BLK
    seg2 = segment.astype(jnp.int32).reshape(N, 1)
    inst2 = instance.astype(jnp.int32).reshape(N, 1)
    row = lambda v: v.reshape(1, -1).astype(f32)

    grid = (nblk,)
    params = pltpu.CompilerParams(dimension_semantics=("arbitrary",))

    h, sum_h, g_mat, segnll, vcnt = pl.pallas_call(
        _pass1_kernel,
        grid=grid,
        in_specs=[
            pl.BlockSpec((BLK, 6), lambda i: (i, 0)),
            pl.BlockSpec((BLK, 1), lambda i: (i, 0)),
            _full((6, C)), _full((1, C)), _full((C, C)), _full((1, C)),
            _full((C, NUM_CLASSES)), _full((1, NUM_CLASSES)),
        ],
        out_specs=[
            pl.BlockSpec((BLK, C), lambda i: (i, 0)),
            _full((1, C)), _full((C, C)), _full((1, 1)), _full((1, 1)),
        ],
        out_shape=[
            jax.ShapeDtypeStruct((N, C), f32),
            jax.ShapeDtypeStruct((1, C), f32),
            jax.ShapeDtypeStruct((C, C), f32),
            jax.ShapeDtypeStruct((1, 1), f32),
            jax.ShapeDtypeStruct((1, 1), f32),
        ],
        compiler_params=params,
    )(feat, seg2, W_b1, row(b_b1), W_b2, row(b_b2), W_seg, row(b_seg))

    l1xy, cosxy, l1z, cosz, cnt = pl.pallas_call(
        _pass2_kernel,
        grid=grid,
        in_specs=[
            pl.BlockSpec((BLK, C), lambda i: (i, 0)),
            pl.BlockSpec((BLK, 3), lambda i: (i, 0)),
            pl.BlockSpec((BLK, 3), lambda i: (i, 0)),
            pl.BlockSpec((BLK, 1), lambda i: (i, 0)),
            _full((1, C)), _full((C, C)),
            _full((C, C)), _full((1, C)), _full((1, C)), _full((1, C)),
            _full((C, 2)), _full((1, 2)),
            _full((C, C)), _full((1, C)), _full((1, C)), _full((1, C)),
            _full((C, 1)), _full((1, 1)),
        ],
        out_specs=[_full((1, 1))] * 5,
        out_shape=[jax.ShapeDtypeStruct((1, 1), f32)] * 5,
        scratch_shapes=[
            pltpu.VMEM((C, C), f32), pltpu.VMEM((1, C), f32),
            pltpu.VMEM((C, C), f32), pltpu.VMEM((1, C), f32),
        ],
        compiler_params=params,
    )(h, coord, instance_centroid, inst2, sum_h, g_mat,
      W_xy1, row(b_xy1), row(g_xy), row(be_xy), W_xy2, row(b_xy2),
      W_z1, row(b_z1), row(g_z), row(be_z), W_z2, row(b_z2))

    seg_loss = segnll[0, 0] / jnp.maximum(vcnt[0, 0], 1.0)
    denom = cnt[0, 0] + 1e-8
    bias_xy_l1 = l1xy[0, 0] / denom
    bias_xy_cos = cosxy[0, 0] / denom
    bias_z_l1 = l1z[0, 0] / denom
    bias_z_cos = cosz[0, 0] / denom
    loss = seg_loss + 2.0 * bias_xy_l1 + 0.5 * (bias_z_l1 + bias_z_cos)
    return jnp.stack([loss, seg_loss, bias_xy_l1, bias_xy_cos,
                      bias_z_l1, bias_z_cos])
